# baseline (speedup 1.0000x reference)
import functools

import jax
import jax.numpy as jnp
from jax import lax
from jax.experimental import pallas as pl
from jax.experimental.pallas import tpu as pltpu
from jax.experimental.pallas import tpu_sc as plsc

N_NODES = 10000
N_EDGES = 320000
D = 128

NC = 2
NS = 16
NW = NC * NS

CK = 125
EPW = N_EDGES // NW
M = EPW // CK

RPT = 624
RPT_LAST = N_NODES - (NS - 1) * RPT


NBUF = 2
MH = 8
NSTAGE = M // MH


def _sc_agg_body(x_hbm, src_hbm, dst_hbm, out_hbm,
                 src_v, dst_v, bufs, acc, gsems, ssems, isems):
    c = lax.axis_index("c")
    s = lax.axis_index("s")
    w = s * NC + c

    row0 = pl.multiple_of(s * RPT, 8)
    pltpu.sync_copy(x_hbm.at[pl.ds(row0, RPT)], acc.at[pl.ds(row0, RPT)])

    @pl.when(s == NS - 1)
    def _():
        tail = NS * RPT
        pltpu.sync_copy(x_hbm.at[pl.ds(tail, N_NODES - NS * RPT)],
                        acc.at[pl.ds(tail, N_NODES - NS * RPT)])

    plsc.subcore_barrier()

    def mk_ops(src_c, dst_c):
        def gather(j, k):
            pltpu.async_copy(x_hbm.at[src_c.at[j]], bufs.at[k], gsems.at[k])

        def gather_wait(j, k):
            pltpu.make_async_copy(x_hbm.at[src_c.at[j]], bufs.at[k],
                                  gsems.at[k]).wait()

        def scatter(j, k):
            pltpu.async_copy(bufs.at[k], acc.at[dst_c.at[j]], ssems.at[k],
                             add=True)

        def scatter_wait(j, k):
            pltpu.make_async_copy(bufs.at[k], acc.at[dst_c.at[j]],
                                  ssems.at[k]).wait()

        return gather, gather_wait, scatter, scatter_wait

    G = MH // NBUF
    R = MH % NBUF
    base = w * M

    def stage(h, slot):
        pltpu.async_copy(src_hbm.at[pl.ds(base + h * MH, MH)],
                         src_v.at[slot], isems.at[slot, 0])
        pltpu.async_copy(dst_hbm.at[pl.ds(base + h * MH, MH)],
                         dst_v.at[slot], isems.at[slot, 1])

    def stage_wait(h, slot):
        pltpu.make_async_copy(src_hbm.at[pl.ds(base + h * MH, MH)],
                              src_v.at[slot], isems.at[slot, 0]).wait()
        pltpu.make_async_copy(dst_hbm.at[pl.ds(base + h * MH, MH)],
                              dst_v.at[slot], isems.at[slot, 1]).wait()

    stage(0, 0)
    for h in range(NSTAGE):
        slot = h % 2
        stage_wait(h, slot)
        if h + 1 < NSTAGE:
            stage(h + 1, (h + 1) % 2)
        gather, gather_wait, scatter, scatter_wait = mk_ops(
            src_v.at[slot], dst_v.at[slot])

        for k in range(NBUF):
            gather(k, k)

        def body(i, carry):
            j0 = i * NBUF
            for k in range(NBUF):
                gather_wait(j0 + k, k)
                scatter(j0 + k, k)
            @pl.when(i < G - 1)
            def _():
                for k in range(NBUF):
                    scatter_wait(j0 + k, k)
                    gather(j0 + NBUF + k, k)
            return carry

        lax.fori_loop(0, G, body, 0)

        last0 = (G - 1) * NBUF
        for r in range(R):
            scatter_wait(last0 + r, r)
            gather(G * NBUF + r, r)
        for r in range(R, NBUF):
            scatter_wait(last0 + r, r)
        for r in range(R):
            gather_wait(G * NBUF + r, r)
            scatter(G * NBUF + r, r)
        for r in range(R):
            scatter_wait(G * NBUF + r, r)

    plsc.subcore_barrier()

    out0 = pl.multiple_of(c * N_NODES + row0, 8)
    pltpu.sync_copy(acc.at[pl.ds(row0, RPT)], out_hbm.at[pl.ds(out0, RPT)])

    @pl.when(s == NS - 1)
    def _():
        tail = NS * RPT
        otail = pl.multiple_of(c * N_NODES + tail, 8)
        pltpu.sync_copy(acc.at[pl.ds(tail, N_NODES - NS * RPT)],
                        out_hbm.at[pl.ds(otail, N_NODES - NS * RPT)])


_sc_agg = functools.partial(
    pl.kernel,
    out_type=jax.ShapeDtypeStruct((2 * N_NODES, D), jnp.float32),
    mesh=plsc.VectorSubcoreMesh(
        core_axis_name="c", subcore_axis_name="s",
        num_cores=NC, num_subcores=NS),
    scratch_types=[
        pltpu.VMEM((2, MH, CK), jnp.int32),
        pltpu.VMEM((2, MH, CK), jnp.int32),
        pltpu.VMEM((NBUF, CK, D), jnp.float32),
        pltpu.VMEM_SHARED((N_NODES, D), jnp.float32),
        pltpu.SemaphoreType.DMA((NBUF,)),
        pltpu.SemaphoreType.DMA((NBUF,)),
        pltpu.SemaphoreType.DMA((2, 2)),
    ],
)(_sc_agg_body)


BLK = 2000


def _tc_body(p_ref, x_ref, w_ref, b_ref, wq_ref, bq_ref, out_ref, acc_ref):
    i = pl.program_id(0)
    tot = p_ref[0] + p_ref[1] - x_ref[...]
    h = jnp.dot(tot, w_ref[...], preferred_element_type=jnp.float32)
    h = jnp.maximum(h + b_ref[...], 0.0)
    part = jnp.sum(h, axis=0, keepdims=True)

    @pl.when(i == 0)
    def _():
        acc_ref[...] = jnp.zeros_like(acc_ref)

    acc_ref[0:1, :] += part

    @pl.when(i == pl.num_programs(0) - 1)
    def _():
        g = acc_ref[0:1, :]
        q = jnp.dot(g, wq_ref[...], preferred_element_type=jnp.float32)
        out_ref[...] = jnp.maximum(q + bq_ref[...], 0.0)


def _tc_head(p, x, w_gnn, b_gnn, wq_pad, bq_pad):
    grid = N_NODES // BLK
    return pl.pallas_call(
        _tc_body,
        grid=(grid,),
        in_specs=[
            pl.BlockSpec((2, BLK, D), lambda i: (0, i, 0)),
            pl.BlockSpec((BLK, D), lambda i: (i, 0)),
            pl.BlockSpec((D, D), lambda i: (0, 0)),
            pl.BlockSpec((1, D), lambda i: (0, 0)),
            pl.BlockSpec((D, D), lambda i: (0, 0)),
            pl.BlockSpec((1, D), lambda i: (0, 0)),
        ],
        out_specs=pl.BlockSpec((1, D), lambda i: (0, 0)),
        out_shape=jax.ShapeDtypeStruct((1, D), jnp.float32),
        scratch_shapes=[pltpu.VMEM((8, D), jnp.float32)],
    )(p, x, w_gnn, b_gnn, wq_pad, bq_pad)


def kernel(x, edge_index, W_gnn, b_gnn, W_q, b_q):
    src = edge_index[0].astype(jnp.int32).reshape(N_EDGES // CK, CK)
    dst = edge_index[1].astype(jnp.int32).reshape(N_EDGES // CK, CK)

    p = _sc_agg(x, src, dst).reshape(2, N_NODES, D)

    wq_pad = jnp.zeros((D, D), jnp.float32).at[:, :4].set(W_q)
    bq_pad = jnp.zeros((1, D), jnp.float32).at[0, :4].set(b_q)
    q = _tc_head(p, x, W_gnn, b_gnn.reshape(1, D), wq_pad, bq_pad)
    return q[:, :4]


# speedup vs baseline: 1.0253x; 1.0253x over previous
import functools

import jax
import jax.numpy as jnp
from jax import lax
from jax.experimental import pallas as pl
from jax.experimental.pallas import tpu as pltpu
from jax.experimental.pallas import tpu_sc as plsc

N_NODES = 10000
N_EDGES = 320000
D = 128

NC = 2
NS = 16
NW = NC * NS

CK = 25
EPW = N_EDGES // NW
M = EPW // CK

RPT = 624
RPT_LAST = N_NODES - (NS - 1) * RPT


NBUF = 7
MH = 40
NSTAGE = M // MH


def _sc_agg_body(x_hbm, src_hbm, dst_hbm, out_hbm,
                 src_v, dst_v, bufs, acc, gsems, ssems, isems):
    c = lax.axis_index("c")
    s = lax.axis_index("s")
    w = s * NC + c

    row0 = pl.multiple_of(s * RPT, 8)
    pltpu.sync_copy(x_hbm.at[pl.ds(row0, RPT)], acc.at[pl.ds(row0, RPT)])

    @pl.when(s == NS - 1)
    def _():
        tail = NS * RPT
        pltpu.sync_copy(x_hbm.at[pl.ds(tail, N_NODES - NS * RPT)],
                        acc.at[pl.ds(tail, N_NODES - NS * RPT)])

    plsc.subcore_barrier()

    def mk_ops(src_c, dst_c):
        def gather(j, k):
            pltpu.async_copy(x_hbm.at[src_c.at[j]], bufs.at[k], gsems.at[k])

        def gather_wait(j, k):
            pltpu.make_async_copy(x_hbm.at[src_c.at[j]], bufs.at[k],
                                  gsems.at[k]).wait()

        def scatter(j, k):
            pltpu.async_copy(bufs.at[k], acc.at[dst_c.at[j]], ssems.at[k],
                             add=True)

        def scatter_wait(j, k):
            pltpu.make_async_copy(bufs.at[k], acc.at[dst_c.at[j]],
                                  ssems.at[k]).wait()

        return gather, gather_wait, scatter, scatter_wait

    G = MH // NBUF
    R = MH % NBUF
    base = w * M

    def stage(h, slot):
        pltpu.async_copy(src_hbm.at[pl.ds(base + h * MH, MH)],
                         src_v.at[slot], isems.at[slot, 0])
        pltpu.async_copy(dst_hbm.at[pl.ds(base + h * MH, MH)],
                         dst_v.at[slot], isems.at[slot, 1])

    def stage_wait(h, slot):
        pltpu.make_async_copy(src_hbm.at[pl.ds(base + h * MH, MH)],
                              src_v.at[slot], isems.at[slot, 0]).wait()
        pltpu.make_async_copy(dst_hbm.at[pl.ds(base + h * MH, MH)],
                              dst_v.at[slot], isems.at[slot, 1]).wait()

    stage(0, 0)
    for h in range(NSTAGE):
        slot = h % 2
        stage_wait(h, slot)
        if h + 1 < NSTAGE:
            stage(h + 1, (h + 1) % 2)
        gather, gather_wait, scatter, scatter_wait = mk_ops(
            src_v.at[slot], dst_v.at[slot])

        for k in range(NBUF):
            gather(k, k)

        def body(i, carry):
            j0 = i * NBUF
            for k in range(NBUF):
                gather_wait(j0 + k, k)
                scatter(j0 + k, k)
            @pl.when(i < G - 1)
            def _():
                for k in range(NBUF):
                    scatter_wait(j0 + k, k)
                    gather(j0 + NBUF + k, k)
            return carry

        lax.fori_loop(0, G, body, 0)

        last0 = (G - 1) * NBUF
        for r in range(R):
            scatter_wait(last0 + r, r)
            gather(G * NBUF + r, r)
        for r in range(R, NBUF):
            scatter_wait(last0 + r, r)
        for r in range(R):
            gather_wait(G * NBUF + r, r)
            scatter(G * NBUF + r, r)
        for r in range(R):
            scatter_wait(G * NBUF + r, r)

    plsc.subcore_barrier()

    out0 = pl.multiple_of(c * N_NODES + row0, 8)
    pltpu.sync_copy(acc.at[pl.ds(row0, RPT)], out_hbm.at[pl.ds(out0, RPT)])

    @pl.when(s == NS - 1)
    def _():
        tail = NS * RPT
        otail = pl.multiple_of(c * N_NODES + tail, 8)
        pltpu.sync_copy(acc.at[pl.ds(tail, N_NODES - NS * RPT)],
                        out_hbm.at[pl.ds(otail, N_NODES - NS * RPT)])


_sc_agg = functools.partial(
    pl.kernel,
    out_type=jax.ShapeDtypeStruct((2 * N_NODES, D), jnp.float32),
    mesh=plsc.VectorSubcoreMesh(
        core_axis_name="c", subcore_axis_name="s",
        num_cores=NC, num_subcores=NS),
    scratch_types=[
        pltpu.VMEM((2, MH, CK), jnp.int32),
        pltpu.VMEM((2, MH, CK), jnp.int32),
        pltpu.VMEM((NBUF, CK, D), jnp.float32),
        pltpu.VMEM_SHARED((N_NODES, D), jnp.float32),
        pltpu.SemaphoreType.DMA((NBUF,)),
        pltpu.SemaphoreType.DMA((NBUF,)),
        pltpu.SemaphoreType.DMA((2, 2)),
    ],
)(_sc_agg_body)


BLK = 2000


def _tc_body(p_ref, x_ref, w_ref, b_ref, wq_ref, bq_ref, out_ref, acc_ref):
    i = pl.program_id(0)
    tot = p_ref[0] + p_ref[1] - x_ref[...]
    h = jnp.dot(tot, w_ref[...], preferred_element_type=jnp.float32)
    h = jnp.maximum(h + b_ref[...], 0.0)
    part = jnp.sum(h, axis=0, keepdims=True)

    @pl.when(i == 0)
    def _():
        acc_ref[...] = jnp.zeros_like(acc_ref)

    acc_ref[0:1, :] += part

    @pl.when(i == pl.num_programs(0) - 1)
    def _():
        g = acc_ref[0:1, :]
        q = jnp.dot(g, wq_ref[...], preferred_element_type=jnp.float32)
        out_ref[...] = jnp.maximum(q + bq_ref[...], 0.0)


def _tc_head(p, x, w_gnn, b_gnn, wq_pad, bq_pad):
    grid = N_NODES // BLK
    return pl.pallas_call(
        _tc_body,
        grid=(grid,),
        in_specs=[
            pl.BlockSpec((2, BLK, D), lambda i: (0, i, 0)),
            pl.BlockSpec((BLK, D), lambda i: (i, 0)),
            pl.BlockSpec((D, D), lambda i: (0, 0)),
            pl.BlockSpec((1, D), lambda i: (0, 0)),
            pl.BlockSpec((D, D), lambda i: (0, 0)),
            pl.BlockSpec((1, D), lambda i: (0, 0)),
        ],
        out_specs=pl.BlockSpec((1, D), lambda i: (0, 0)),
        out_shape=jax.ShapeDtypeStruct((1, D), jnp.float32),
        scratch_shapes=[pltpu.VMEM((8, D), jnp.float32)],
    )(p, x, w_gnn, b_gnn, wq_pad, bq_pad)


def kernel(x, edge_index, W_gnn, b_gnn, W_q, b_q):
    src = edge_index[0].astype(jnp.int32).reshape(N_EDGES // CK, CK)
    dst = edge_index[1].astype(jnp.int32).reshape(N_EDGES // CK, CK)

    p = _sc_agg(x, src, dst).reshape(2, N_NODES, D)

    wq_pad = jnp.zeros((D, D), jnp.float32).at[:, :4].set(W_q)
    bq_pad = jnp.zeros((1, D), jnp.float32).at[0, :4].set(b_q)
    q = _tc_head(p, x, W_gnn, b_gnn.reshape(1, D), wq_pad, bq_pad)
    return q[:, :4]


# speedup vs baseline: 1.1365x; 1.1085x over previous
import functools

import jax
import jax.numpy as jnp
from jax import lax
from jax.experimental import pallas as pl
from jax.experimental.pallas import tpu as pltpu
from jax.experimental.pallas import tpu_sc as plsc

N_NODES = 10000
N_EDGES = 320000
D = 128

NC = 2
NS = 16
NW = NC * NS

CK = 50
EPW = N_EDGES // NW
M = EPW // CK

RPT = 624
RPT_LAST = N_NODES - (NS - 1) * RPT


NBUF = 4
MH = 40
NSTAGE = M // MH


def _sc_agg_body(x_hbm, src_hbm, dst_hbm, out_hbm,
                 src_v, dst_v, bufs, acc, gsems, ssems, isems):
    c = lax.axis_index("c")
    s = lax.axis_index("s")
    w = s * NC + c

    row0 = pl.multiple_of(s * RPT, 8)
    pltpu.sync_copy(x_hbm.at[pl.ds(row0, RPT)], acc.at[pl.ds(row0, RPT)])

    @pl.when(s == NS - 1)
    def _():
        tail = NS * RPT
        pltpu.sync_copy(x_hbm.at[pl.ds(tail, N_NODES - NS * RPT)],
                        acc.at[pl.ds(tail, N_NODES - NS * RPT)])

    plsc.subcore_barrier()

    def mk_ops(src_c, dst_c):
        def gather(j, k):
            pltpu.async_copy(x_hbm.at[src_c.at[j]], bufs.at[k], gsems.at[k])

        def gather_wait(j, k):
            pltpu.make_async_copy(x_hbm.at[src_c.at[j]], bufs.at[k],
                                  gsems.at[k]).wait()

        def scatter(j, k):
            pltpu.async_copy(bufs.at[k], acc.at[dst_c.at[j]], ssems.at[k],
                             add=True)

        def scatter_wait(j, k):
            pltpu.make_async_copy(bufs.at[k], acc.at[dst_c.at[j]],
                                  ssems.at[k]).wait()

        return gather, gather_wait, scatter, scatter_wait

    G = MH // NBUF
    R = MH % NBUF
    base = w * M

    def stage(h, slot):
        pltpu.async_copy(src_hbm.at[pl.ds(base + h * MH, MH)],
                         src_v.at[slot], isems.at[slot, 0])
        pltpu.async_copy(dst_hbm.at[pl.ds(base + h * MH, MH)],
                         dst_v.at[slot], isems.at[slot, 1])

    def stage_wait(h, slot):
        pltpu.make_async_copy(src_hbm.at[pl.ds(base + h * MH, MH)],
                              src_v.at[slot], isems.at[slot, 0]).wait()
        pltpu.make_async_copy(dst_hbm.at[pl.ds(base + h * MH, MH)],
                              dst_v.at[slot], isems.at[slot, 1]).wait()

    stage(0, 0)
    for h in range(NSTAGE):
        slot = h % 2
        stage_wait(h, slot)
        if h + 1 < NSTAGE:
            stage(h + 1, (h + 1) % 2)
        gather, gather_wait, scatter, scatter_wait = mk_ops(
            src_v.at[slot], dst_v.at[slot])

        for k in range(NBUF):
            gather(k, k)

        def body(i, carry):
            j0 = i * NBUF
            for k in range(NBUF):
                gather_wait(j0 + k, k)
                scatter(j0 + k, k)
            @pl.when(i < G - 1)
            def _():
                for k in range(NBUF):
                    scatter_wait(j0 + k, k)
                    gather(j0 + NBUF + k, k)
            return carry

        lax.fori_loop(0, G, body, 0)

        last0 = (G - 1) * NBUF
        for r in range(R):
            scatter_wait(last0 + r, r)
            gather(G * NBUF + r, r)
        for r in range(R, NBUF):
            scatter_wait(last0 + r, r)
        for r in range(R):
            gather_wait(G * NBUF + r, r)
            scatter(G * NBUF + r, r)
        for r in range(R):
            scatter_wait(G * NBUF + r, r)

    plsc.subcore_barrier()

    out0 = pl.multiple_of(c * N_NODES + row0, 8)
    pltpu.sync_copy(acc.at[pl.ds(row0, RPT)], out_hbm.at[pl.ds(out0, RPT)])

    @pl.when(s == NS - 1)
    def _():
        tail = NS * RPT
        otail = pl.multiple_of(c * N_NODES + tail, 8)
        pltpu.sync_copy(acc.at[pl.ds(tail, N_NODES - NS * RPT)],
                        out_hbm.at[pl.ds(otail, N_NODES - NS * RPT)])


_sc_agg = functools.partial(
    pl.kernel,
    out_type=jax.ShapeDtypeStruct((2 * N_NODES, D), jnp.float32),
    mesh=plsc.VectorSubcoreMesh(
        core_axis_name="c", subcore_axis_name="s",
        num_cores=NC, num_subcores=NS),
    scratch_types=[
        pltpu.VMEM((2, MH, CK), jnp.int32),
        pltpu.VMEM((2, MH, CK), jnp.int32),
        pltpu.VMEM((NBUF, CK, D), jnp.float32),
        pltpu.VMEM_SHARED((N_NODES, D), jnp.float32),
        pltpu.SemaphoreType.DMA((NBUF,)),
        pltpu.SemaphoreType.DMA((NBUF,)),
        pltpu.SemaphoreType.DMA((2, 2)),
    ],
)(_sc_agg_body)


BLK = 5000


def _tc_body(p_ref, x_ref, w_ref, b_ref, wq_ref, bq_ref, out_ref, acc_ref):
    i = pl.program_id(0)
    tot = p_ref[0] + p_ref[1] - x_ref[...]
    h = jnp.dot(tot, w_ref[...], preferred_element_type=jnp.float32)
    h = jnp.maximum(h + b_ref[...], 0.0)
    part = jnp.sum(h, axis=0, keepdims=True)

    @pl.when(i == 0)
    def _():
        acc_ref[...] = jnp.zeros_like(acc_ref)

    acc_ref[0:1, :] += part

    @pl.when(i == pl.num_programs(0) - 1)
    def _():
        g = acc_ref[0:1, :]
        q = jnp.dot(g, wq_ref[...], preferred_element_type=jnp.float32)
        out_ref[...] = jnp.maximum(q + bq_ref[...], 0.0)


def _tc_head(p, x, w_gnn, b_gnn, wq_pad, bq_pad):
    grid = N_NODES // BLK
    return pl.pallas_call(
        _tc_body,
        grid=(grid,),
        in_specs=[
            pl.BlockSpec((2, BLK, D), lambda i: (0, i, 0)),
            pl.BlockSpec((BLK, D), lambda i: (i, 0)),
            pl.BlockSpec((D, D), lambda i: (0, 0)),
            pl.BlockSpec((1, D), lambda i: (0, 0)),
            pl.BlockSpec((D, D), lambda i: (0, 0)),
            pl.BlockSpec((1, D), lambda i: (0, 0)),
        ],
        out_specs=pl.BlockSpec((1, D), lambda i: (0, 0)),
        out_shape=jax.ShapeDtypeStruct((1, D), jnp.float32),
        scratch_shapes=[pltpu.VMEM((8, D), jnp.float32)],
    )(p, x, w_gnn, b_gnn, wq_pad, bq_pad)


def kernel(x, edge_index, W_gnn, b_gnn, W_q, b_q):
    src = edge_index[0].astype(jnp.int32).reshape(N_EDGES // CK, CK)
    dst = edge_index[1].astype(jnp.int32).reshape(N_EDGES // CK, CK)

    p = _sc_agg(x, src, dst).reshape(2, N_NODES, D)

    wq_pad = jnp.zeros((D, D), jnp.float32).at[:, :4].set(W_q)
    bq_pad = jnp.zeros((1, D), jnp.float32).at[0, :4].set(b_q)
    q = _tc_head(p, x, W_gnn, b_gnn.reshape(1, D), wq_pad, bq_pad)
    return q[:, :4]
